# baseline (speedup 1.0000x reference)
import jax
import jax.numpy as jnp
from jax.experimental import pallas as pl
from jax.experimental.pallas import tpu as pltpu

WIDTH = 128
N_REL = 16
N_NODE = 512
HALF = N_NODE // 2
REL_PER_STEP = 2
STEPS = N_REL // REL_PER_STEP


def _body(x_ref, dc0, dr0, dc1, dr1, a1w_ref, a2w_ref, opw_ref, cw_ref,
          out_ref, arg1_s, arg2_s, x2b_s, acc_s, cwsm_s):
    c = pl.program_id(0)
    r = pl.program_id(1)

    @pl.when(r == 0)
    def _init():
        x = x_ref[...]
        w1 = a1w_ref[...]
        w1 = jnp.exp(w1 - jnp.max(w1, axis=0, keepdims=True))
        w1 = w1 / jnp.sum(w1, axis=0, keepdims=True)
        w2 = a2w_ref[...]
        w2 = jnp.exp(w2 - jnp.max(w2, axis=0, keepdims=True))
        w2 = w2 / jnp.sum(w2, axis=0, keepdims=True)
        arg1_s[...] = jax.lax.dot_general(
            w1, x, (((0,), (0,)), ((), ())), preferred_element_type=jnp.float32)
        a2v = jax.lax.dot_general(
            w2, x, (((0,), (0,)), ((), ())), preferred_element_type=jnp.float32)
        arg2_s[...] = a2v
        x2b_s[...] = a2v.astype(jnp.bfloat16)
        cw = cw_ref[...]
        cw = jnp.exp(cw - jnp.max(cw, axis=1, keepdims=True))
        cwsm_s[...] = cw / jnp.sum(cw, axis=1, keepdims=True)
        acc_s[...] = jnp.zeros_like(acc_s)

    x2b = x2b_s[...]
    cwsm = cwsm_s[...]
    lane = jax.lax.broadcasted_iota(jnp.int32, (WIDTH, 2 * N_REL), 1)
    acc = acc_s[...]
    for j, (dc, dr) in enumerate(((dc0, dr0), (dc1, dr1))):
        rel = REL_PER_STEP * r + j
        fwd = jax.lax.dot_general(
            x2b, dc[0].astype(jnp.bfloat16), (((1,), (0,)), ((), ())),
            preferred_element_type=jnp.float32)
        bwd = jax.lax.dot_general(
            x2b, dr[0].astype(jnp.bfloat16), (((1,), (1,)), ((), ())),
            preferred_element_type=jnp.float32)
        w1c = jnp.sum(jnp.where(lane == rel, cwsm, 0.0), axis=1, keepdims=True)
        w2c = jnp.sum(jnp.where(lane == rel + N_REL, cwsm, 0.0),
                      axis=1, keepdims=True)
        acc = acc + w1c * fwd + w2c * bwd
    acc_s[...] = acc

    @pl.when(r == STEPS - 1)
    def _finish():
        chain = 1.0 - jnp.exp(-acc)
        opw = opw_ref[...]
        opw = jnp.exp(opw - jnp.max(opw, axis=1, keepdims=True))
        opw = opw / jnp.sum(opw, axis=1, keepdims=True)
        a1f = arg1_s[...]
        a2f = arg2_s[...]
        a1 = jnp.where(c == 0, a1f[:, :HALF], a1f[:, HALF:])
        a2 = jnp.where(c == 0, a2f[:, :HALF], a2f[:, HALF:])
        a12 = a1 * a2
        out_ref[...] = (opw[:, 0:1] * a2
                        + opw[:, 1:2] * a12
                        + opw[:, 2:3] * (a1 + a2 - a12)
                        + opw[:, 3:4] * chain
                        + opw[:, 4:5] * (1.0 - a1))


def kernel(inputs, database, arg1_weights, arg2_weights, op_weights, chain_weights):
    dbr = database.reshape(2 * N_REL, HALF, N_NODE)
    return pl.pallas_call(
        _body,
        grid=(2, STEPS),
        in_specs=[
            pl.BlockSpec((WIDTH, N_NODE), lambda c, r: (0, 0)),
            pl.BlockSpec((1, N_NODE, HALF), lambda c, r: (2 * r, 0, c)),
            pl.BlockSpec((1, HALF, N_NODE), lambda c, r: (4 * r + c, 0, 0)),
            pl.BlockSpec((1, N_NODE, HALF), lambda c, r: (2 * r + 1, 0, c)),
            pl.BlockSpec((1, HALF, N_NODE), lambda c, r: (4 * r + 2 + c, 0, 0)),
            pl.BlockSpec((WIDTH, WIDTH), lambda c, r: (0, 0)),
            pl.BlockSpec((WIDTH, WIDTH), lambda c, r: (0, 0)),
            pl.BlockSpec((WIDTH, len(op_weights[0])), lambda c, r: (0, 0)),
            pl.BlockSpec((WIDTH, 2 * N_REL), lambda c, r: (0, 0)),
        ],
        out_specs=pl.BlockSpec((WIDTH, HALF), lambda c, r: (0, c)),
        out_shape=jax.ShapeDtypeStruct((WIDTH, N_NODE), jnp.float32),
        scratch_shapes=[
            pltpu.VMEM((WIDTH, N_NODE), jnp.float32),
            pltpu.VMEM((WIDTH, N_NODE), jnp.float32),
            pltpu.VMEM((WIDTH, N_NODE), jnp.bfloat16),
            pltpu.VMEM((WIDTH, HALF), jnp.float32),
            pltpu.VMEM((WIDTH, 2 * N_REL), jnp.float32),
        ],
        compiler_params=pltpu.CompilerParams(
            dimension_semantics=("parallel", "arbitrary")),
    )(inputs, database, dbr, database, dbr,
      arg1_weights, arg2_weights, op_weights, chain_weights)


# speedup vs baseline: 1.8899x; 1.8899x over previous
import jax
import jax.numpy as jnp
from jax.experimental import pallas as pl
from jax.experimental.pallas import tpu as pltpu

WIDTH = 128
N_REL = 16
N_NODE = 512


def _body(x_ref, db_hbm, a1w_ref, a2w_ref, opw_ref, cw_ref,
          out_ref, dbv, sems):
    copies = [
        pltpu.make_async_copy(db_hbm.at[i], dbv.at[i], sems.at[i])
        for i in range(N_REL)
    ]
    for c in copies:
        c.start()

    x = x_ref[...]
    w1 = a1w_ref[...]
    w1 = jnp.exp(w1 - jnp.max(w1, axis=0, keepdims=True))
    w1 = w1 / jnp.sum(w1, axis=0, keepdims=True)
    w2 = a2w_ref[...]
    w2 = jnp.exp(w2 - jnp.max(w2, axis=0, keepdims=True))
    w2 = w2 / jnp.sum(w2, axis=0, keepdims=True)
    arg1 = jax.lax.dot_general(
        w1, x, (((0,), (0,)), ((), ())), preferred_element_type=jnp.float32)
    arg2 = jax.lax.dot_general(
        w2, x, (((0,), (0,)), ((), ())), preferred_element_type=jnp.float32)
    cw = cw_ref[...]
    cw = jnp.exp(cw - jnp.max(cw, axis=1, keepdims=True))
    cwsm = cw / jnp.sum(cw, axis=1, keepdims=True)

    opw = opw_ref[...]
    opw = jnp.exp(opw - jnp.max(opw, axis=1, keepdims=True))
    opw = opw / jnp.sum(opw, axis=1, keepdims=True)

    acc = jnp.zeros((WIDTH, N_NODE), jnp.float32)
    for i in range(N_REL):
        copies[i].wait()
        d = dbv[i]
        xw1 = arg2 * cwsm[:, i:i + 1]
        xw2 = arg2 * cwsm[:, N_REL + i:N_REL + i + 1]
        fwd = jax.lax.dot_general(
            xw1, d, (((1,), (0,)), ((), ())),
            precision=jax.lax.Precision.DEFAULT,
            preferred_element_type=jnp.float32)
        bwd = jax.lax.dot_general(
            xw2, d, (((1,), (1,)), ((), ())),
            precision=jax.lax.Precision.DEFAULT,
            preferred_element_type=jnp.float32)
        acc = acc + fwd + bwd

    chain = 1.0 - jnp.exp(-acc)
    a12 = arg1 * arg2
    out_ref[...] = (opw[:, 0:1] * arg2
                    + opw[:, 1:2] * a12
                    + opw[:, 2:3] * (arg1 + arg2 - a12)
                    + opw[:, 3:4] * chain
                    + opw[:, 4:5] * (1.0 - arg1))


def kernel(inputs, database, arg1_weights, arg2_weights, op_weights, chain_weights):
    return pl.pallas_call(
        _body,
        in_specs=[
            pl.BlockSpec(memory_space=pltpu.MemorySpace.VMEM),
            pl.BlockSpec(memory_space=pltpu.MemorySpace.HBM),
            pl.BlockSpec(memory_space=pltpu.MemorySpace.VMEM),
            pl.BlockSpec(memory_space=pltpu.MemorySpace.VMEM),
            pl.BlockSpec(memory_space=pltpu.MemorySpace.VMEM),
            pl.BlockSpec(memory_space=pltpu.MemorySpace.VMEM),
        ],
        out_specs=pl.BlockSpec(memory_space=pltpu.MemorySpace.VMEM),
        out_shape=jax.ShapeDtypeStruct((WIDTH, N_NODE), jnp.float32),
        scratch_shapes=[
            pltpu.VMEM((N_REL, N_NODE, N_NODE), jnp.float32),
            pltpu.SemaphoreType.DMA((N_REL,)),
        ],
    )(inputs, database, arg1_weights, arg2_weights, op_weights, chain_weights)


# speedup vs baseline: 1.9198x; 1.0158x over previous
import jax
import jax.numpy as jnp
from jax.experimental import pallas as pl
from jax.experimental.pallas import tpu as pltpu

WIDTH = 128
N_REL = 16
N_NODE = 512


def _body(x_ref, db_hbm, a1w_ref, a2w_ref, opw_ref, cw_ref,
          out_ref, dbv, sems):
    copies = [
        pltpu.make_async_copy(db_hbm.at[i], dbv.at[i], sems.at[i])
        for i in range(N_REL)
    ]
    for c in copies:
        c.start()

    x = x_ref[...]
    w1 = a1w_ref[...]
    w1 = jnp.exp(w1 - jnp.max(w1, axis=0, keepdims=True))
    w1 = w1 / jnp.sum(w1, axis=0, keepdims=True)
    w2 = a2w_ref[...]
    w2 = jnp.exp(w2 - jnp.max(w2, axis=0, keepdims=True))
    w2 = w2 / jnp.sum(w2, axis=0, keepdims=True)
    arg1 = jax.lax.dot_general(
        w1, x, (((0,), (0,)), ((), ())), preferred_element_type=jnp.float32)
    arg2 = jax.lax.dot_general(
        w2, x, (((0,), (0,)), ((), ())), preferred_element_type=jnp.float32)
    cw = cw_ref[...]
    cw = jnp.exp(cw - jnp.max(cw, axis=1, keepdims=True))
    cwsm = cw / jnp.sum(cw, axis=1, keepdims=True)

    opw = opw_ref[...]
    opw = jnp.exp(opw - jnp.max(opw, axis=1, keepdims=True))
    opw = opw / jnp.sum(opw, axis=1, keepdims=True)

    acc = jnp.zeros((WIDTH, N_NODE), jnp.float32)
    for i in range(N_REL):
        copies[i].wait()
        d = dbv[i].astype(jnp.bfloat16)
        xw1 = (arg2 * cwsm[:, i:i + 1]).astype(jnp.bfloat16)
        xw2 = (arg2 * cwsm[:, N_REL + i:N_REL + i + 1]).astype(jnp.bfloat16)
        fwd = jax.lax.dot_general(
            xw1, d, (((1,), (0,)), ((), ())),
            preferred_element_type=jnp.float32)
        bwd = jax.lax.dot_general(
            xw2, d, (((1,), (1,)), ((), ())),
            preferred_element_type=jnp.float32)
        acc = acc + fwd + bwd

    chain = 1.0 - jnp.exp(-acc)
    a12 = arg1 * arg2
    out_ref[...] = (opw[:, 0:1] * arg2
                    + opw[:, 1:2] * a12
                    + opw[:, 2:3] * (arg1 + arg2 - a12)
                    + opw[:, 3:4] * chain
                    + opw[:, 4:5] * (1.0 - arg1))


def kernel(inputs, database, arg1_weights, arg2_weights, op_weights, chain_weights):
    return pl.pallas_call(
        _body,
        in_specs=[
            pl.BlockSpec(memory_space=pltpu.MemorySpace.VMEM),
            pl.BlockSpec(memory_space=pltpu.MemorySpace.HBM),
            pl.BlockSpec(memory_space=pltpu.MemorySpace.VMEM),
            pl.BlockSpec(memory_space=pltpu.MemorySpace.VMEM),
            pl.BlockSpec(memory_space=pltpu.MemorySpace.VMEM),
            pl.BlockSpec(memory_space=pltpu.MemorySpace.VMEM),
        ],
        out_specs=pl.BlockSpec(memory_space=pltpu.MemorySpace.VMEM),
        out_shape=jax.ShapeDtypeStruct((WIDTH, N_NODE), jnp.float32),
        scratch_shapes=[
            pltpu.VMEM((N_REL, N_NODE, N_NODE), jnp.float32),
            pltpu.SemaphoreType.DMA((N_REL,)),
        ],
    )(inputs, database, arg1_weights, arg2_weights, op_weights, chain_weights)
